# baseline (speedup 1.0000x reference)
import functools

import jax
import jax.numpy as jnp
from jax.experimental import pallas as pl
from jax.experimental.pallas import tpu as pltpu

B, T, N_EMBED = 4, 2048, 1024
NUM_EXPERTS, TOP_K = 16, 2
TILE = 512
N_TOKENS = B * T


def _router_body(x_ref, wt_ref, swt_ref, b_ref, sb_ref, ei_ref, eo_ref,
                 rout_ref, idx_ref, z_ref):
    i = pl.program_id(0)
    ei = ei_ref[:]
    eo = eo_ref[:]
    fei = jnp.sign(ei) * jnp.sqrt(jnp.abs(ei))
    feo = jnp.sign(eo) * jnp.sqrt(jnp.abs(eo))
    nw = wt_ref[:] + swt_ref[:] * (fei * feo)
    nb = b_ref[:] + sb_ref[:] * feo
    logits = jnp.dot(x_ref[:], nw, preferred_element_type=jnp.float32) + nb

    iota = jax.lax.broadcasted_iota(jnp.int32, logits.shape, 1)
    m1 = jnp.max(logits, axis=1, keepdims=True)
    i1 = jnp.min(jnp.where(logits == m1, iota, NUM_EXPERTS), axis=1,
                 keepdims=True)
    sel1 = iota == i1
    masked = jnp.where(sel1, -jnp.inf, logits)
    m2 = jnp.max(masked, axis=1, keepdims=True)
    i2 = jnp.min(jnp.where(masked == m2, iota, NUM_EXPERTS), axis=1,
                 keepdims=True)
    sel2 = iota == i2

    e = jnp.exp(m2 - m1)
    denom = 1.0 + e
    p1 = 1.0 / denom
    p2 = e / denom
    rout_ref[:] = jnp.where(sel1, p1, jnp.where(sel2, p2, 0.0))
    idx_ref[:, 0:1] = i1
    idx_ref[:, 1:2] = i2

    lse = m1 + jnp.log1p(e)
    part = jnp.sum(lse * lse)

    @pl.when(i == 0)
    def _init():
        z_ref[0, 0] = part

    @pl.when(i != 0)
    def _acc():
        z_ref[0, 0] += part


@jax.jit
def _router(x, wt, swt, b2, sb2, ei2, eo2):
    grid = N_TOKENS // TILE
    rout, idx, zsum = pl.pallas_call(
        _router_body,
        grid=(grid,),
        in_specs=[
            pl.BlockSpec((TILE, N_EMBED), lambda i: (i, 0)),
            pl.BlockSpec((N_EMBED, NUM_EXPERTS), lambda i: (0, 0)),
            pl.BlockSpec((N_EMBED, NUM_EXPERTS), lambda i: (0, 0)),
            pl.BlockSpec((1, NUM_EXPERTS), lambda i: (0, 0)),
            pl.BlockSpec((1, NUM_EXPERTS), lambda i: (0, 0)),
            pl.BlockSpec((N_EMBED, 1), lambda i: (0, 0)),
            pl.BlockSpec((1, NUM_EXPERTS), lambda i: (0, 0)),
        ],
        out_specs=[
            pl.BlockSpec((TILE, NUM_EXPERTS), lambda i: (i, 0)),
            pl.BlockSpec((TILE, 2), lambda i: (i, 0)),
            pl.BlockSpec(memory_space=pltpu.SMEM),
        ],
        out_shape=[
            jax.ShapeDtypeStruct((N_TOKENS, NUM_EXPERTS), jnp.float32),
            jax.ShapeDtypeStruct((N_TOKENS, 2), jnp.int32),
            jax.ShapeDtypeStruct((1, 1), jnp.float32),
        ],
    )(x, wt, swt, b2, sb2, ei2, eo2)
    return rout, idx, zsum


def kernel(mh_output, W, sigma_W, b, sigma_b, eps_in, eps_out):
    x = mh_output.reshape(N_TOKENS, N_EMBED)
    wt = W.T
    swt = sigma_W.T
    b2 = b.reshape(1, NUM_EXPERTS)
    sb2 = sigma_b.reshape(1, NUM_EXPERTS)
    ei2 = eps_in.reshape(N_EMBED, 1)
    eo2 = eps_out.reshape(1, NUM_EXPERTS)
    rout, idx, zsum = _router(x, wt, swt, b2, sb2, ei2, eo2)
    router_output = rout.reshape(B, T, NUM_EXPERTS)
    indices = idx.reshape(B, T, TOP_K)
    z_loss = zsum[0, 0] / jnp.float32(N_TOKENS)
    return router_output, indices, z_loss


# speedup vs baseline: 1.2462x; 1.2462x over previous
import functools

import jax
import jax.numpy as jnp
from jax.experimental import pallas as pl
from jax.experimental.pallas import tpu as pltpu

B, T, N_EMBED = 4, 2048, 1024
NUM_EXPERTS, TOP_K = 16, 2
TILE = 2048
N_TOKENS = B * T


def _router_body(x_ref, wt_ref, swt_ref, b_ref, sb_ref, ei_ref, eo_ref,
                 rout_ref, idx_ref, z_ref):
    i = pl.program_id(0)
    ei = ei_ref[:]
    eo = eo_ref[:]
    fei = jnp.sign(ei) * jnp.sqrt(jnp.abs(ei))
    feo = jnp.sign(eo) * jnp.sqrt(jnp.abs(eo))
    nw = wt_ref[:] + swt_ref[:] * (fei * feo)
    nb = b_ref[:] + sb_ref[:] * feo
    logits = jnp.dot(x_ref[:], nw, preferred_element_type=jnp.float32) + nb

    iota = jax.lax.broadcasted_iota(jnp.int32, logits.shape, 1)
    m1 = jnp.max(logits, axis=1, keepdims=True)
    i1 = jnp.min(jnp.where(logits == m1, iota, NUM_EXPERTS), axis=1,
                 keepdims=True)
    sel1 = iota == i1
    masked = jnp.where(sel1, -jnp.inf, logits)
    m2 = jnp.max(masked, axis=1, keepdims=True)
    i2 = jnp.min(jnp.where(masked == m2, iota, NUM_EXPERTS), axis=1,
                 keepdims=True)
    sel2 = iota == i2

    e = jnp.exp(m2 - m1)
    denom = 1.0 + e
    p1 = 1.0 / denom
    p2 = e / denom
    rout_ref[:] = jnp.where(sel1, p1, jnp.where(sel2, p2, 0.0))
    idx_ref[:, 0:1] = i1
    idx_ref[:, 1:2] = i2

    lse = m1 + jnp.log1p(e)
    part = jnp.sum(lse * lse)

    @pl.when(i == 0)
    def _init():
        z_ref[0, 0] = part

    @pl.when(i != 0)
    def _acc():
        z_ref[0, 0] += part


@jax.jit
def _router(x, wt, swt, b2, sb2, ei2, eo2):
    grid = N_TOKENS // TILE
    rout, idx, zsum = pl.pallas_call(
        _router_body,
        grid=(grid,),
        in_specs=[
            pl.BlockSpec((TILE, N_EMBED), lambda i: (i, 0)),
            pl.BlockSpec((N_EMBED, NUM_EXPERTS), lambda i: (0, 0)),
            pl.BlockSpec((N_EMBED, NUM_EXPERTS), lambda i: (0, 0)),
            pl.BlockSpec((1, NUM_EXPERTS), lambda i: (0, 0)),
            pl.BlockSpec((1, NUM_EXPERTS), lambda i: (0, 0)),
            pl.BlockSpec((N_EMBED, 1), lambda i: (0, 0)),
            pl.BlockSpec((1, NUM_EXPERTS), lambda i: (0, 0)),
        ],
        out_specs=[
            pl.BlockSpec((TILE, NUM_EXPERTS), lambda i: (i, 0)),
            pl.BlockSpec((TILE, 2), lambda i: (i, 0)),
            pl.BlockSpec(memory_space=pltpu.SMEM),
        ],
        out_shape=[
            jax.ShapeDtypeStruct((N_TOKENS, NUM_EXPERTS), jnp.float32),
            jax.ShapeDtypeStruct((N_TOKENS, 2), jnp.int32),
            jax.ShapeDtypeStruct((1, 1), jnp.float32),
        ],
    )(x, wt, swt, b2, sb2, ei2, eo2)
    return rout, idx, zsum


def kernel(mh_output, W, sigma_W, b, sigma_b, eps_in, eps_out):
    x = mh_output.reshape(N_TOKENS, N_EMBED)
    wt = W.T
    swt = sigma_W.T
    b2 = b.reshape(1, NUM_EXPERTS)
    sb2 = sigma_b.reshape(1, NUM_EXPERTS)
    ei2 = eps_in.reshape(N_EMBED, 1)
    eo2 = eps_out.reshape(1, NUM_EXPERTS)
    rout, idx, zsum = _router(x, wt, swt, b2, sb2, ei2, eo2)
    router_output = rout.reshape(B, T, NUM_EXPERTS)
    indices = idx.reshape(B, T, TOP_K)
    z_loss = zsum[0, 0] / jnp.float32(N_TOKENS)
    return router_output, indices, z_loss


# speedup vs baseline: 1.9748x; 1.5847x over previous
import functools

import jax
import jax.numpy as jnp
from jax.experimental import pallas as pl
from jax.experimental.pallas import tpu as pltpu

B, T, N_EMBED = 4, 2048, 1024
NUM_EXPERTS, TOP_K = 16, 2
TILE = 2048
N_TOKENS = B * T


def _router_body(x_ref, w_ref, sw_ref, b_ref, sb_ref, ei_ref, eo_ref,
                 rout_ref, idx_ref, z_ref):
    i = pl.program_id(0)
    ei = ei_ref[:]
    eo = eo_ref[:]
    fei = jnp.sign(ei) * jnp.sqrt(jnp.abs(ei))
    feo = jnp.sign(eo) * jnp.sqrt(jnp.abs(eo))
    nw = w_ref[:] + sw_ref[:] * (feo * fei)
    nb = b_ref[:] + sb_ref[:] * feo
    logits = jax.lax.dot_general(
        nw, x_ref[:], (((1,), (1,)), ((), ())),
        preferred_element_type=jnp.float32) + nb

    iota = jax.lax.broadcasted_iota(jnp.int32, logits.shape, 0)
    m1 = jnp.max(logits, axis=0, keepdims=True)
    i1 = jnp.min(jnp.where(logits == m1, iota, NUM_EXPERTS), axis=0,
                 keepdims=True)
    sel1 = iota == i1
    masked = jnp.where(sel1, -jnp.inf, logits)
    m2 = jnp.max(masked, axis=0, keepdims=True)
    i2 = jnp.min(jnp.where(masked == m2, iota, NUM_EXPERTS), axis=0,
                 keepdims=True)
    sel2 = iota == i2

    e = jnp.exp(m2 - m1)
    denom = 1.0 + e
    p1 = 1.0 / denom
    p2 = e / denom
    rout_ref[:] = jnp.where(sel1, p1, jnp.where(sel2, p2, 0.0))
    idx_ref[0:1, :] = i1
    idx_ref[1:2, :] = i2

    lse = m1 + jnp.log1p(e)
    part = jnp.sum(lse * lse)

    @pl.when(i == 0)
    def _init():
        z_ref[0, 0] = part

    @pl.when(i != 0)
    def _acc():
        z_ref[0, 0] += part


@jax.jit
def _router(x, w, sw, b2, sb2, ei2, eo2):
    grid = N_TOKENS // TILE
    routT, idxT, zsum = pl.pallas_call(
        _router_body,
        grid=(grid,),
        in_specs=[
            pl.BlockSpec((TILE, N_EMBED), lambda i: (i, 0)),
            pl.BlockSpec((NUM_EXPERTS, N_EMBED), lambda i: (0, 0)),
            pl.BlockSpec((NUM_EXPERTS, N_EMBED), lambda i: (0, 0)),
            pl.BlockSpec((NUM_EXPERTS, 1), lambda i: (0, 0)),
            pl.BlockSpec((NUM_EXPERTS, 1), lambda i: (0, 0)),
            pl.BlockSpec((1, N_EMBED), lambda i: (0, 0)),
            pl.BlockSpec((NUM_EXPERTS, 1), lambda i: (0, 0)),
        ],
        out_specs=[
            pl.BlockSpec((NUM_EXPERTS, TILE), lambda i: (0, i)),
            pl.BlockSpec((2, TILE), lambda i: (0, i)),
            pl.BlockSpec(memory_space=pltpu.SMEM),
        ],
        out_shape=[
            jax.ShapeDtypeStruct((NUM_EXPERTS, N_TOKENS), jnp.float32),
            jax.ShapeDtypeStruct((2, N_TOKENS), jnp.int32),
            jax.ShapeDtypeStruct((1, 1), jnp.float32),
        ],
    )(x, w, sw, b2, sb2, ei2, eo2)
    return routT, idxT, zsum


def kernel(mh_output, W, sigma_W, b, sigma_b, eps_in, eps_out):
    x = mh_output.reshape(N_TOKENS, N_EMBED)
    b2 = b.reshape(NUM_EXPERTS, 1)
    sb2 = sigma_b.reshape(NUM_EXPERTS, 1)
    ei2 = eps_in.reshape(1, N_EMBED)
    eo2 = eps_out.reshape(NUM_EXPERTS, 1)
    routT, idxT, zsum = _router(x, W, sigma_W, b2, sb2, ei2, eo2)
    router_output = routT.T.reshape(B, T, NUM_EXPERTS)
    indices = idxT.T.reshape(B, T, TOP_K)
    z_loss = zsum[0, 0] / jnp.float32(N_TOKENS)
    return router_output, indices, z_loss
